# Initial kernel scaffold; baseline (speedup 1.0000x reference)
import jax
import jax.numpy as jnp
from jax.experimental import pallas as pl


def kernel(input, token_table, pos_table):
    raise NotImplementedError("write your pallas kernel here")


# speedup vs baseline: 1.2788x; 1.2788x over previous
import functools

import jax
import jax.numpy as jnp
from jax import lax
from jax.experimental import pallas as pl
from jax.experimental.pallas import tpu as pltpu
from jax.experimental.pallas import tpu_sc as plsc

_LANES = 16
_IDX_COLS = 128


def kernel(input, token_table, pos_table):
    B, S = input.shape
    V, D = token_table.shape
    N = B * S

    info = plsc.get_sparse_core_info()
    NW = info.num_cores * info.num_subcores
    RPW = N // NW
    IDX_ROWS_PER_W = RPW // _IDX_COLS

    flat_idx = input.reshape(N // _IDX_COLS, _IDX_COLS).astype(jnp.int32)

    mesh = plsc.VectorSubcoreMesh(core_axis_name="c", subcore_axis_name="s")

    @functools.partial(
        pl.kernel,
        out_type=jax.ShapeDtypeStruct((N, D), jnp.float32),
        mesh=mesh,
        scratch_types=[
            pltpu.VMEM((IDX_ROWS_PER_W, _IDX_COLS), jnp.int32),
            pltpu.VMEM((RPW, D), jnp.float32),
            pltpu.VMEM((RPW, D), jnp.float32),
            pltpu.SemaphoreType.DMA,
            pltpu.SemaphoreType.DMA,
        ],
    )
    def sc_embed(idx_hbm, tok_hbm, pos_hbm, out_hbm, idx_v, tok_v, pos_v,
                 gsem, psem):
        wid = lax.axis_index("s") * info.num_cores + lax.axis_index("c")
        base = wid * RPW
        pos_base = lax.rem(base, S)

        pltpu.sync_copy(
            idx_hbm.at[pl.ds(wid * IDX_ROWS_PER_W, IDX_ROWS_PER_W)], idx_v)
        pos_copy = pltpu.async_copy(
            pos_hbm.at[pl.ds(pos_base, RPW)], pos_v, psem)
        gathers = [
            pltpu.async_copy(
                tok_hbm.at[idx_v.at[j]],
                tok_v.at[pl.ds(j * _IDX_COLS, _IDX_COLS)],
                gsem,
            )
            for j in range(IDX_ROWS_PER_W)
        ]
        pos_copy.wait()
        for g in gathers:
            g.wait()

        def row_body(r, carry):
            for c in range(D // _LANES):
                sl = pl.ds(c * _LANES, _LANES)
                plsc.addupdate(tok_v.at[r, sl], pos_v[r, sl])
            return carry

        lax.fori_loop(0, RPW, row_body, 0)

        pltpu.sync_copy(tok_v, out_hbm.at[pl.ds(base, RPW)])

    out = sc_embed(flat_idx, token_table, pos_table)
    return out.reshape(B, S, D)


# speedup vs baseline: 1.3050x; 1.0205x over previous
import functools

import jax
import jax.numpy as jnp
from jax import lax
from jax.experimental import pallas as pl
from jax.experimental.pallas import tpu as pltpu
from jax.experimental.pallas import tpu_sc as plsc

_LANES = 16


def kernel(input, token_table, pos_table):
    B, S = input.shape
    V, D = token_table.shape
    N = B * S

    info = plsc.get_sparse_core_info()
    NW = info.num_cores * info.num_subcores
    PPW = S // NW
    CD = D // _LANES

    idx32 = input.astype(jnp.int32)

    mesh = plsc.VectorSubcoreMesh(core_axis_name="c", subcore_axis_name="s")

    @functools.partial(
        pl.kernel,
        out_type=jax.ShapeDtypeStruct((N, D), jnp.float32),
        mesh=mesh,
        scratch_types=[
            pltpu.VMEM((B, PPW), jnp.int32),
            pltpu.VMEM((B * PPW, D), jnp.float32),
            pltpu.VMEM((PPW, D), jnp.float32),
            [pltpu.SemaphoreType.DMA] * B,
            pltpu.SemaphoreType.DMA,
            pltpu.SemaphoreType.DMA,
        ],
    )
    def sc_embed(idx_hbm, tok_hbm, pos_hbm, out_hbm, idx_v, tok_v, pos_v,
                 gsems, psem, ssem):
        wid = lax.axis_index("s") * info.num_cores + lax.axis_index("c")
        p0 = wid * PPW

        for b in range(B):
            pltpu.sync_copy(idx_hbm.at[b, pl.ds(p0, PPW)], idx_v.at[b])
        pos_copy = pltpu.async_copy(pos_hbm.at[pl.ds(p0, PPW)], pos_v, psem)
        gathers = [
            pltpu.async_copy(
                tok_hbm.at[idx_v.at[b]],
                tok_v.at[pl.ds(b * PPW, PPW)],
                gsems[b],
            )
            for b in range(B)
        ]
        pos_copy.wait()
        for g in gathers:
            g.wait()

        def row_body(r, carry):
            for c in range(CD):
                sl = pl.ds(c * _LANES, _LANES)
                pvec = pos_v[r, sl]
                for b in range(B):
                    plsc.addupdate(tok_v.at[b * PPW + r, sl], pvec)
            return carry

        lax.fori_loop(0, PPW, row_body, 0)

        stores = [
            pltpu.async_copy(
                tok_v.at[pl.ds(b * PPW, PPW)],
                out_hbm.at[pl.ds(b * S + p0, PPW)],
                ssem,
            )
            for b in range(B)
        ]
        for st in stores:
            st.wait()

    out = sc_embed(idx32, token_table, pos_table)
    return out.reshape(B, S, D)


# speedup vs baseline: 1.4016x; 1.0740x over previous
import functools

import jax
import jax.numpy as jnp
from jax import lax
from jax.experimental import pallas as pl
from jax.experimental.pallas import tpu as pltpu
from jax.experimental.pallas import tpu_sc as plsc

_LANES = 16


def kernel(input, token_table, pos_table):
    B, S = input.shape
    V, D = token_table.shape
    N = B * S

    info = plsc.get_sparse_core_info()
    NW = info.num_cores * info.num_subcores
    PPW = S // NW
    CD = D // _LANES

    idx32 = input.astype(jnp.int32)

    H = 2
    GP = PPW // H

    mesh = plsc.VectorSubcoreMesh(core_axis_name="c", subcore_axis_name="s")

    @functools.partial(
        pl.kernel,
        out_type=jax.ShapeDtypeStruct((N, D), jnp.float32),
        mesh=mesh,
        scratch_types=[
            pltpu.VMEM((B, PPW), jnp.int32),
            pltpu.VMEM((B * PPW, D), jnp.float32),
            pltpu.VMEM((PPW, D), jnp.float32),
            [pltpu.SemaphoreType.DMA] * H,
            pltpu.SemaphoreType.DMA,
            pltpu.SemaphoreType.DMA,
            pltpu.SemaphoreType.DMA,
        ],
    )
    def sc_embed(idx_hbm, tok_hbm, pos_hbm, out_hbm, idx_v, tok_v, pos_v,
                 gsems, isem, psem, ssem):
        wid = lax.axis_index("s") * info.num_cores + lax.axis_index("c")
        p0 = wid * PPW

        idx_copies = [
            pltpu.async_copy(idx_hbm.at[b, pl.ds(p0, PPW)], idx_v.at[b], isem)
            for b in range(B)
        ]
        pos_copy = pltpu.async_copy(
            pos_hbm.at[pl.ds(p0, PPW)], pos_v, psem)
        for c in idx_copies:
            c.wait()
        gathers = [
            [
                pltpu.async_copy(
                    tok_hbm.at[idx_v.at[b, pl.ds(h * GP, GP)]],
                    tok_v.at[pl.ds(b * PPW + h * GP, GP)],
                    gsems[h],
                )
                for b in range(B)
            ]
            for h in range(H)
        ]
        pos_copy.wait()

        stores = []
        for h in range(H):
            for g in gathers[h]:
                g.wait()

            def row_body(r, carry, _h=h):
                for c in range(CD):
                    sl = pl.ds(c * _LANES, _LANES)
                    pvec = pos_v[_h * GP + r, sl]
                    for b in range(B):
                        plsc.addupdate(tok_v.at[b * PPW + _h * GP + r, sl],
                                       pvec)
                return carry

            lax.fori_loop(0, GP, row_body, 0)

            for b in range(B):
                stores.append(pltpu.async_copy(
                    tok_v.at[pl.ds(b * PPW + h * GP, GP)],
                    out_hbm.at[pl.ds(b * S + p0 + h * GP, GP)],
                    ssem,
                ))

        for st in stores:
            st.wait()

    out = sc_embed(idx32, token_table, pos_table)
    return out.reshape(B, S, D)


# speedup vs baseline: 1.4066x; 1.0035x over previous
import functools

import jax
import jax.numpy as jnp
from jax import lax
from jax.experimental import pallas as pl
from jax.experimental.pallas import tpu as pltpu
from jax.experimental.pallas import tpu_sc as plsc

_LANES = 16


def kernel(input, token_table, pos_table):
    B, S = input.shape
    V, D = token_table.shape
    N = B * S

    info = plsc.get_sparse_core_info()
    NW = info.num_cores * info.num_subcores
    PPW = S // NW
    CD = D // _LANES

    idx32 = input.astype(jnp.int32)

    H = 2
    GP = PPW // H

    mesh = plsc.VectorSubcoreMesh(core_axis_name="c", subcore_axis_name="s")

    @functools.partial(
        pl.kernel,
        out_type=jax.ShapeDtypeStruct((N, D), jnp.float32),
        mesh=mesh,
        scratch_types=[
            pltpu.VMEM((B, PPW), jnp.int32),
            pltpu.VMEM((B * PPW, D), jnp.float32),
            pltpu.VMEM((PPW, D), jnp.float32),
            [pltpu.SemaphoreType.DMA] * H,
            [pltpu.SemaphoreType.DMA] * B,
            pltpu.SemaphoreType.DMA,
            pltpu.SemaphoreType.DMA,
        ],
    )
    def sc_embed(idx_hbm, tok_hbm, pos_hbm, out_hbm, idx_v, tok_v, pos_v,
                 gsems, isems, psem, ssem):
        wid = lax.axis_index("s") * info.num_cores + lax.axis_index("c")
        p0 = wid * PPW

        idx_copies = [
            pltpu.async_copy(idx_hbm.at[b, pl.ds(p0, PPW)], idx_v.at[b],
                             isems[b])
            for b in range(B)
        ]
        pos_copy = pltpu.async_copy(
            pos_hbm.at[pl.ds(p0, PPW)], pos_v, psem)
        gathers = [[None] * B for _ in range(H)]
        for b in range(B):
            idx_copies[b].wait()
            for h in range(H):
                gathers[h][b] = pltpu.async_copy(
                    tok_hbm.at[idx_v.at[b, pl.ds(h * GP, GP)]],
                    tok_v.at[pl.ds(b * PPW + h * GP, GP)],
                    gsems[h],
                )
        pos_copy.wait()

        stores = []
        for h in range(H):
            for g in gathers[h]:
                g.wait()

            def row_body(r, carry, _h=h):
                for c in range(CD):
                    sl = pl.ds(c * _LANES, _LANES)
                    pvec = pos_v[_h * GP + r, sl]
                    for b in range(B):
                        plsc.addupdate(tok_v.at[b * PPW + _h * GP + r, sl],
                                       pvec)
                return carry

            lax.fori_loop(0, GP, row_body, 0)

            for b in range(B):
                stores.append(pltpu.async_copy(
                    tok_v.at[pl.ds(b * PPW + h * GP, GP)],
                    out_hbm.at[pl.ds(b * S + p0 + h * GP, GP)],
                    ssem,
                ))

        for st in stores:
            st.wait()

    out = sc_embed(idx32, token_table, pos_table)
    return out.reshape(B, S, D)


# speedup vs baseline: 1.4114x; 1.0034x over previous
import functools

import jax
import jax.numpy as jnp
from jax import lax
from jax.experimental import pallas as pl
from jax.experimental.pallas import tpu as pltpu
from jax.experimental.pallas import tpu_sc as plsc

_LANES = 16


def kernel(input, token_table, pos_table):
    B, S = input.shape
    V, D = token_table.shape
    N = B * S

    info = plsc.get_sparse_core_info()
    NW = info.num_cores * info.num_subcores
    PPW = S // NW
    CD = D // _LANES

    idx32 = input.astype(jnp.int32)

    GROUPS = [(0, 8), (8, 24), (32, 32)]
    assert sum(n for _, n in GROUPS) == PPW
    H = len(GROUPS)

    mesh = plsc.VectorSubcoreMesh(core_axis_name="c", subcore_axis_name="s")

    @functools.partial(
        pl.kernel,
        out_type=jax.ShapeDtypeStruct((N, D), jnp.float32),
        mesh=mesh,
        scratch_types=[
            pltpu.VMEM((B, PPW), jnp.int32),
            pltpu.VMEM((B * PPW, D), jnp.float32),
            pltpu.VMEM((PPW, D), jnp.float32),
            [pltpu.SemaphoreType.DMA] * H,
            [pltpu.SemaphoreType.DMA] * B,
            [pltpu.SemaphoreType.DMA] * 2,
            pltpu.SemaphoreType.DMA,
        ],
    )
    def sc_embed(idx_hbm, tok_hbm, pos_hbm, out_hbm, idx_v, tok_v, pos_v,
                 gsems, isems, psems, ssem):
        wid = lax.axis_index("s") * info.num_cores + lax.axis_index("c")
        p0 = wid * PPW
        g0 = GROUPS[0][1]

        idx_copies = [
            pltpu.async_copy(idx_hbm.at[b, pl.ds(p0, PPW)], idx_v.at[b],
                             isems[b])
            for b in range(B)
        ]
        pos_copies = [
            pltpu.async_copy(pos_hbm.at[pl.ds(p0, g0)],
                             pos_v.at[pl.ds(0, g0)], psems[0]),
            pltpu.async_copy(pos_hbm.at[pl.ds(p0 + g0, PPW - g0)],
                             pos_v.at[pl.ds(g0, PPW - g0)], psems[1]),
        ]
        gathers = [[None] * B for _ in range(H)]
        for b in range(B):
            idx_copies[b].wait()
            for h, (off, n) in enumerate(GROUPS):
                gathers[h][b] = pltpu.async_copy(
                    tok_hbm.at[idx_v.at[b, pl.ds(off, n)]],
                    tok_v.at[pl.ds(b * PPW + off, n)],
                    gsems[h],
                )

        stores = []
        for h, (off, n) in enumerate(GROUPS):
            if h < 2:
                pos_copies[h].wait()
            for g in gathers[h]:
                g.wait()

            def row_body(r, carry, _off=off):
                for c in range(CD):
                    sl = pl.ds(c * _LANES, _LANES)
                    pvec = pos_v[_off + r, sl]
                    for b in range(B):
                        plsc.addupdate(tok_v.at[b * PPW + _off + r, sl],
                                       pvec)
                return carry

            lax.fori_loop(0, n, row_body, 0)

            for b in range(B):
                stores.append(pltpu.async_copy(
                    tok_v.at[pl.ds(b * PPW + off, n)],
                    out_hbm.at[pl.ds(b * S + p0 + off, n)],
                    ssem,
                ))

        for st in stores:
            st.wait()

    out = sc_embed(idx32, token_table, pos_table)
    return out.reshape(B, S, D)
